# baseline (speedup 1.0000x reference)
import functools

import jax
import jax.numpy as jnp
from jax import lax
from jax.experimental import pallas as pl
from jax.experimental.pallas import tpu as pltpu
from jax.experimental.pallas import tpu_sc as plsc

NC = 2
NS = 16
NW = NC * NS
CW = 128
ACW = 80
SCH = 8
DB = 70

_mesh = functools.partial(
    plsc.VectorSubcoreMesh, core_axis_name="c", subcore_axis_name="s")
_params = pltpu.CompilerParams(needs_layout_passes=False,
                               use_tc_tiling_on_sc=False)
_params_tc = pltpu.CompilerParams(needs_layout_passes=False)


def _deg_body(npad, ch, dst_hbm, ew_hbm, out_hbm, dst_v, ew_v, zv, deg_sp):
  c = lax.axis_index("c")
  s = lax.axis_index("s")
  wid = s * NC + c
  nsl = npad // NS

  def zbody(i, _):
    zv[pl.ds(i * 16, 16)] = jnp.zeros((16,), jnp.float32)
    return ()

  lax.fori_loop(0, nsl // 16, zbody, ())
  pltpu.sync_copy(zv, deg_sp.at[pl.ds(s * nsl, nsl)])
  pltpu.sync_copy(dst_hbm.at[wid], dst_v)
  pltpu.sync_copy(ew_hbm.at[wid], ew_v)
  plsc.subcore_barrier()

  def body(k, _):
    pltpu.sync_copy(ew_v.at[k], deg_sp.at[dst_v.at[k]], add=True)
    return ()

  lax.fori_loop(0, ch, body, ())
  plsc.subcore_barrier()
  pltpu.sync_copy(deg_sp.at[pl.ds(s * nsl, nsl)], zv)
  pltpu.sync_copy(zv, out_hbm.at[pl.ds(c * npad + s * nsl, nsl)])


def _agg_body(n, nacc, npad, ch, d, src_hbm, dst_hbm, ew_hbm, dinv_hbm,
              xw_hbm, out_hbm, src_v, dst_v, ew_v, norm_v, dinv_v,
              g0, g1, g2, s0, s1, acc_sp,
              ga, gb, gc, sa, sb_):
  c = lax.axis_index("c")
  s = lax.axis_index("s")
  wid = s * NC + c
  rsl = nacc // NS
  gbufs = (g0, g1, g2)
  sbufs = (s0, s1)
  gsems = (ga, gb, gc)
  ssems = (sa, sb_)

  def zrow(r, _):
    for j in range(d // 16):
      s0[r, pl.ds(j * 16, 16)] = jnp.zeros((16,), jnp.float32)
    return ()

  lax.fori_loop(0, DB, zrow, ())
  for b in range(rsl // DB):
    pltpu.sync_copy(s0.at[pl.ds(0, DB)],
                    acc_sp.at[pl.ds(s * rsl + b * DB, DB)])
  pltpu.sync_copy(dinv_hbm, dinv_v)
  plsc.subcore_barrier()

  def scale_rows(gi, si, k):
    gbuf = gbufs[gi]
    sbuf = sbufs[si]

    def row(r, _):
      nb = plsc.load_gather(
          norm_v, [jnp.full((16,), k, jnp.int32),
                   jnp.full((16,), r, jnp.int32)])
      for g in range(d // 32):
        pk = gbuf[r, pl.ds(g * 32, 32)]
        lo, hi = plsc.unpack(pk, format=plsc.PackFormat.INTERLEAVED)
        sbuf[r, pl.ds(g * 32, 16)] = lo * nb
        sbuf[r, pl.ds(g * 32 + 16, 16)] = hi * nb
      return ()

    lax.fori_loop(0, ACW, row, ())

  def gat(k, b):
    return pltpu.make_async_copy(xw_hbm.at[src_v.at[k]], gbufs[b],
                                 gsems[b])

  def scat(k, b):
    return pltpu.make_async_copy(sbufs[b], acc_sp.at[dst_v.at[k]],
                                 ssems[b])

  def super_chunk(sb, _):
    pltpu.sync_copy(src_hbm.at[wid, pl.ds(sb * SCH, SCH)], src_v)
    pltpu.sync_copy(dst_hbm.at[wid, pl.ds(sb * SCH, SCH)], dst_v)
    pltpu.sync_copy(ew_hbm.at[wid, pl.ds(sb * SCH, SCH)], ew_v)

    def norm_row(r, _):
      for j in range(ACW // 16):
        sidx = src_v[r, pl.ds(j * 16, 16)]
        didx = dst_v[r, pl.ds(j * 16, 16)]
        dv_s = plsc.load_gather(dinv_v, [sidx])
        dv_d = plsc.load_gather(dinv_v, [didx])
        norm_v[r, pl.ds(j * 16, 16)] = (
            dv_s * ew_v[r, pl.ds(j * 16, 16)] * dv_d)
      return ()

    lax.fori_loop(0, SCH, norm_row, ())

    gat(0, 0).start()
    gat(1, 1).start()
    for k in range(SCH):
      gi = k % 3
      si = k % 2
      gat(k, gi).wait()
      if k >= 2:
        scat(k - 2, si).wait()
      scale_rows(gi, si, k)
      scat(k, si).start(add=True)
      if k + 2 < SCH:
        gat(k + 2, (k + 2) % 3).start()
    scat(SCH - 2, SCH % 2).wait()
    scat(SCH - 1, (SCH - 1) % 2).wait()
    return ()

  lax.fori_loop(0, ch // SCH, super_chunk, ())
  plsc.subcore_barrier()
  for b in range(rsl // DB):
    buf = sbufs[b % 2]
    pltpu.sync_copy(acc_sp.at[pl.ds(s * rsl + b * DB, DB)],
                    buf.at[pl.ds(0, DB)])
    pltpu.sync_copy(buf.at[pl.ds(0, DB)],
                    out_hbm.at[c, pl.ds(s * rsl + b * DB, DB)])


def _make_deg(npad, ch):
  return pl.kernel(
      functools.partial(_deg_body, npad, ch),
      out_type=jax.ShapeDtypeStruct((NC * npad,), jnp.float32),
      mesh=_mesh(),
      compiler_params=_params,
      scratch_types=[
          pltpu.VMEM((ch, CW), jnp.int32),
          pltpu.VMEM((ch, CW), jnp.float32),
          pltpu.VMEM((npad // NS,), jnp.float32),
          pltpu.VMEM_SHARED((npad,), jnp.float32),
      ])


def _make_agg(n, nacc, npad, ch, d):
  return pl.kernel(
      functools.partial(_agg_body, n, nacc, npad, ch, d),
      out_type=jax.ShapeDtypeStruct((NC, nacc, d), jnp.float32),
      mesh=_mesh(),
      compiler_params=_params,
      scratch_types=[
          pltpu.VMEM((SCH, ACW), jnp.int32),
          pltpu.VMEM((SCH, ACW), jnp.int32),
          pltpu.VMEM((SCH, ACW), jnp.float32),
          pltpu.VMEM((SCH, ACW), jnp.float32),
          pltpu.VMEM((npad,), jnp.float32),
          pltpu.VMEM((ACW, d), jnp.bfloat16),
          pltpu.VMEM((ACW, d), jnp.bfloat16),
          pltpu.VMEM((ACW, d), jnp.bfloat16),
          pltpu.VMEM((ACW, d), jnp.float32),
          pltpu.VMEM((ACW, d), jnp.float32),
          pltpu.VMEM_SHARED((nacc, d), jnp.float32),
          pltpu.SemaphoreType.DMA,
          pltpu.SemaphoreType.DMA,
          pltpu.SemaphoreType.DMA,
          pltpu.SemaphoreType.DMA,
          pltpu.SemaphoreType.DMA,
      ])


def _dinv_tc(degp_ref, dinv_ref, sl_ref):
  deg = 1.0 + degp_ref[0] + degp_ref[1]
  dinv = jnp.where(deg > 0, lax.rsqrt(jnp.where(deg > 0, deg, 1.0)), 0.0)
  dinv_ref[...] = dinv
  sl_ref[...] = dinv * dinv


def _xw_tc(x_ref, w_ref, out_ref):
  out_ref[...] = lax.dot_general(
      x_ref[...], w_ref[...], (((1,), (0,)), ((), ())),
      preferred_element_type=jnp.float32)


def _epi_mm_tc(acc_ref, xw_ref, sl_ref, b_ref, w_ref, a_ref, h_ref, xw2_ref):
  a = a_ref[0, 0]
  h = acc_ref[0] + acc_ref[1] + xw_ref[...] * sl_ref[...] + b_ref[...]
  h = jnp.where(h >= 0, h, a * h)
  h_ref[...] = h
  xw2_ref[...] = lax.dot_general(
      h, w_ref[...], (((1,), (0,)), ((), ())),
      preferred_element_type=jnp.float32)


def _epi_tc(acc_ref, xw_ref, sl_ref, b_ref, a_ref, h_ref):
  a = a_ref[0, 0]
  h = acc_ref[0] + acc_ref[1] + xw_ref[...] * sl_ref[...] + b_ref[...]
  h_ref[...] = jnp.where(h >= 0, h, a * h)


def kernel(x, edge_index, edge_weight, W1, b1, W2, b2, a):
  n, d = x.shape
  e = edge_weight.shape[0]
  src = edge_index[0].astype(jnp.int32)
  dst = edge_index[1].astype(jnp.int32)
  ew = edge_weight.astype(jnp.float32)

  def padded(arrs, lanes, pad_to):
    epw = -(-e // (NW * pad_to)) * pad_to
    pad = epw * NW - e
    pidx = (jnp.arange(pad, dtype=jnp.int32) * 131) % n
    pz = jnp.zeros((pad,), jnp.float32)
    out = []
    for arr in arrs:
      fill = pz if arr.dtype == jnp.float32 else pidx
      out.append(jnp.concatenate([arr, fill])
                 .reshape(NW, epw // lanes, lanes))
    return out, epw // lanes

  (dst_p, ew_p), chd = padded([dst, ew], CW, CW)
  (src_a, dst_a, ew_a), ch = padded([src, dst, ew], ACW, ACW * SCH)

  npad = -(-n // 256) * 256
  a2d = jnp.reshape(a, (1, 1)).astype(jnp.float32)

  degp = _make_deg(npad, chd)(dst_p, ew_p)
  degp2 = degp.reshape(NC, npad // 128, 128)
  dinv2d, sl2d = pl.pallas_call(
      _dinv_tc,
      out_shape=[jax.ShapeDtypeStruct((npad // 128, 128), jnp.float32),
                 jax.ShapeDtypeStruct((npad // 128, 128), jnp.float32)],
  )(degp2)
  dinv = dinv2d.reshape(npad)
  sl_n = sl2d.reshape(npad, 1)

  bn = 1000
  grid = n // bn
  mm = pl.pallas_call(
      _xw_tc,
      grid=(grid,),
      in_specs=[pl.BlockSpec((bn, d), lambda i: (i, 0)),
                pl.BlockSpec((d, d), lambda i: (0, 0))],
      out_specs=pl.BlockSpec((bn, d), lambda i: (i, 0)),
      out_shape=jax.ShapeDtypeStruct((n, d), jnp.float32),
  )
  xw1 = mm(x, W1)

  def to_bf16_interleaved(xw):
    return (xw.reshape(n, d // 32, 2, 16).transpose(0, 1, 3, 2)
            .reshape(n, d).astype(jnp.bfloat16))

  nacc = -(-n // (NS * DB)) * (NS * DB)
  agg = _make_agg(n, nacc, npad, ch, d)
  acc1 = agg(src_a, dst_a, ew_a, dinv, to_bf16_interleaved(xw1))

  epi_mm = pl.pallas_call(
      _epi_mm_tc,
      grid=(grid,),
      in_specs=[pl.BlockSpec((NC, bn, d), lambda i: (0, i, 0)),
                pl.BlockSpec((bn, d), lambda i: (i, 0)),
                pl.BlockSpec((bn, 1), lambda i: (i, 0)),
                pl.BlockSpec((1, d), lambda i: (0, 0)),
                pl.BlockSpec((d, d), lambda i: (0, 0)),
                pl.BlockSpec((1, 1), lambda i: (0, 0))],
      out_specs=[pl.BlockSpec((bn, d), lambda i: (i, 0)),
                 pl.BlockSpec((bn, d), lambda i: (i, 0))],
      out_shape=[jax.ShapeDtypeStruct((n, d), jnp.float32),
                 jax.ShapeDtypeStruct((n, d), jnp.float32)],
  )
  h1, xw2 = epi_mm(acc1, xw1, sl_n, b1.reshape(1, d), W2, a2d)

  acc2 = agg(src_a, dst_a, ew_a, dinv, to_bf16_interleaved(xw2))

  epi = pl.pallas_call(
      _epi_tc,
      grid=(grid,),
      in_specs=[pl.BlockSpec((NC, bn, d), lambda i: (0, i, 0)),
                pl.BlockSpec((bn, d), lambda i: (i, 0)),
                pl.BlockSpec((bn, 1), lambda i: (i, 0)),
                pl.BlockSpec((1, d), lambda i: (0, 0)),
                pl.BlockSpec((1, 1), lambda i: (0, 0))],
      out_specs=pl.BlockSpec((bn, d), lambda i: (i, 0)),
      out_shape=jax.ShapeDtypeStruct((n, d), jnp.float32),
  )
  h2 = epi(acc2, xw2, sl_n, b2.reshape(1, d), a2d)
  return h1, h2


# speedup vs baseline: 1.8222x; 1.8222x over previous
import functools

import jax
import jax.numpy as jnp
from jax import lax
from jax.experimental import pallas as pl
from jax.experimental.pallas import tpu as pltpu
from jax.experimental.pallas import tpu_sc as plsc

NC = 2
NS = 16
NW = NC * NS
CW = 128
ACW = 80
SCH = 12

_mesh = functools.partial(
    plsc.VectorSubcoreMesh, core_axis_name="c", subcore_axis_name="s")
_params = pltpu.CompilerParams(needs_layout_passes=False,
                               use_tc_tiling_on_sc=False)
_params_tc = pltpu.CompilerParams(needs_layout_passes=False)


def _deg_body(npad, ch, dst_hbm, ew_hbm, out_hbm, dst_v, ew_v, zv, deg_sp):
  c = lax.axis_index("c")
  s = lax.axis_index("s")
  wid = s * NC + c
  nsl = npad // NS

  def zbody(i, _):
    zv[pl.ds(i * 16, 16)] = jnp.zeros((16,), jnp.float32)
    return ()

  lax.fori_loop(0, nsl // 16, zbody, ())
  pltpu.sync_copy(zv, deg_sp.at[pl.ds(s * nsl, nsl)])
  pltpu.sync_copy(dst_hbm.at[wid], dst_v)
  pltpu.sync_copy(ew_hbm.at[wid], ew_v)
  plsc.subcore_barrier()

  def body(k, _):
    pltpu.sync_copy(ew_v.at[k], deg_sp.at[dst_v.at[k]], add=True)
    return ()

  lax.fori_loop(0, ch, body, ())
  plsc.subcore_barrier()
  pltpu.sync_copy(deg_sp.at[pl.ds(s * nsl, nsl)], zv)
  pltpu.sync_copy(zv, out_hbm.at[pl.ds(c * npad + s * nsl, nsl)])


def _agg_body(n, npad, ch, d, src_hbm, dst_hbm, ew_hbm, dinv_hbm, xw_hbm,
              out_hbm, src_v, dst_v, ew_v, norm_v, dinv_v,
              rows_a, rows_b, rows_c, acc_sp,
              ga, gb, gc, sa, sb_, sc_):
  c = lax.axis_index("c")
  s = lax.axis_index("s")
  wid = s * NC + c
  rsl = npad // NS
  bufs = (rows_a, rows_b, rows_c)
  gsems = (ga, gb, gc)
  ssems = (sa, sb_, sc_)

  def zrow(r, _):
    for j in range(d // 16):
      rows_a[r, pl.ds(j * 16, 16)] = jnp.zeros((16,), jnp.float32)
    return ()

  lax.fori_loop(0, ACW, zrow, ())
  for b in range(rsl // ACW):
    pltpu.sync_copy(rows_a, acc_sp.at[pl.ds(s * rsl + b * ACW, ACW)])
  pltpu.sync_copy(dinv_hbm, dinv_v)
  plsc.subcore_barrier()

  def scale_rows(buf, k):
    k16 = jnp.full((16,), k, jnp.int32)

    def row(i, _):
      r = 2 * i
      nb0 = plsc.load_gather(norm_v, [k16, jnp.full((16,), r, jnp.int32)])
      nb1 = plsc.load_gather(
          norm_v, [k16, jnp.full((16,), r + 1, jnp.int32)])
      for j in range(d // 16):
        sl = pl.ds(j * 16, 16)
        buf[r, sl] = buf[r, sl] * nb0
      for j in range(d // 16):
        sl = pl.ds(j * 16, 16)
        buf[r + 1, sl] = buf[r + 1, sl] * nb1
      return ()

    lax.fori_loop(0, ACW // 2, row, ())

  def gat(k, b):
    return pltpu.make_async_copy(xw_hbm.at[src_v.at[k]], bufs[b], gsems[b])

  def scat(k, b):
    return pltpu.make_async_copy(bufs[b], acc_sp.at[dst_v.at[k]], ssems[b])

  def super_chunk(sb, _):
    pltpu.sync_copy(src_hbm.at[wid, pl.ds(sb * SCH, SCH)], src_v)
    pltpu.sync_copy(dst_hbm.at[wid, pl.ds(sb * SCH, SCH)], dst_v)
    pltpu.sync_copy(ew_hbm.at[wid, pl.ds(sb * SCH, SCH)], ew_v)

    def norm_row(r, _):
      for j in range(ACW // 16):
        sidx = src_v[r, pl.ds(j * 16, 16)]
        didx = dst_v[r, pl.ds(j * 16, 16)]
        dv_s = plsc.load_gather(dinv_v, [sidx])
        dv_d = plsc.load_gather(dinv_v, [didx])
        norm_v[r, pl.ds(j * 16, 16)] = (
            dv_s * ew_v[r, pl.ds(j * 16, 16)] * dv_d)
      return ()

    lax.fori_loop(0, SCH, norm_row, ())

    gat(0, 0).start()
    gat(1, 1).start()
    for k in range(SCH):
      this = k % 3
      other = (k + 2) % 3
      gat(k, this).wait()
      scale_rows(bufs[this], k)
      if k >= 1:
        scat(k - 1, other).wait()
      if k + 2 < SCH:
        gat(k + 2, other).start()
      scat(k, this).start(add=True)
    scat(SCH - 1, (SCH - 1) % 3).wait()
    return ()

  lax.fori_loop(0, ch // SCH, super_chunk, ())
  plsc.subcore_barrier()
  for b in range(rsl // ACW):
    buf = bufs[b % 3]
    pltpu.sync_copy(acc_sp.at[pl.ds(s * rsl + b * ACW, ACW)], buf)
    pltpu.sync_copy(buf, out_hbm.at[c, pl.ds(s * rsl + b * ACW, ACW)])


def _make_deg(npad, ch):
  return pl.kernel(
      functools.partial(_deg_body, npad, ch),
      out_type=jax.ShapeDtypeStruct((NC * npad,), jnp.float32),
      mesh=_mesh(),
      compiler_params=_params,
      scratch_types=[
          pltpu.VMEM((ch, CW), jnp.int32),
          pltpu.VMEM((ch, CW), jnp.float32),
          pltpu.VMEM((npad // NS,), jnp.float32),
          pltpu.VMEM_SHARED((npad,), jnp.float32),
      ])


def _make_agg(n, npad, ch, d):
  return pl.kernel(
      functools.partial(_agg_body, n, npad, ch, d),
      out_type=jax.ShapeDtypeStruct((NC, npad, d), jnp.float32),
      mesh=_mesh(),
      compiler_params=_params,
      scratch_types=[
          pltpu.VMEM((SCH, ACW), jnp.int32),
          pltpu.VMEM((SCH, ACW), jnp.int32),
          pltpu.VMEM((SCH, ACW), jnp.float32),
          pltpu.VMEM((SCH, ACW), jnp.float32),
          pltpu.VMEM((npad,), jnp.float32),
          pltpu.VMEM((ACW, d), jnp.float32),
          pltpu.VMEM((ACW, d), jnp.float32),
          pltpu.VMEM((ACW, d), jnp.float32),
          pltpu.VMEM_SHARED((npad, d), jnp.float32),
          pltpu.SemaphoreType.DMA,
          pltpu.SemaphoreType.DMA,
          pltpu.SemaphoreType.DMA,
          pltpu.SemaphoreType.DMA,
          pltpu.SemaphoreType.DMA,
          pltpu.SemaphoreType.DMA,
      ])


def _dinv_tc(degp_ref, dinv_ref, sl_ref):
  deg = 1.0 + degp_ref[0] + degp_ref[1]
  dinv = jnp.where(deg > 0, lax.rsqrt(jnp.where(deg > 0, deg, 1.0)), 0.0)
  dinv_ref[...] = dinv
  sl_ref[...] = dinv * dinv


def _xw_tc(x_ref, w_ref, out_ref):
  out_ref[...] = lax.dot_general(
      x_ref[...], w_ref[...], (((1,), (0,)), ((), ())),
      preferred_element_type=jnp.float32)


def _epi_mm_tc(acc_ref, xw_ref, sl_ref, b_ref, w_ref, a_ref, h_ref, xw2_ref):
  a = a_ref[0, 0]
  h = acc_ref[0] + acc_ref[1] + xw_ref[...] * sl_ref[...] + b_ref[...]
  h = jnp.where(h >= 0, h, a * h)
  h_ref[...] = h
  xw2_ref[...] = lax.dot_general(
      h, w_ref[...], (((1,), (0,)), ((), ())),
      preferred_element_type=jnp.float32)


def _epi_tc(acc_ref, xw_ref, sl_ref, b_ref, a_ref, h_ref):
  a = a_ref[0, 0]
  h = acc_ref[0] + acc_ref[1] + xw_ref[...] * sl_ref[...] + b_ref[...]
  h_ref[...] = jnp.where(h >= 0, h, a * h)


def kernel(x, edge_index, edge_weight, W1, b1, W2, b2, a):
  n, d = x.shape
  e = edge_weight.shape[0]
  src = edge_index[0].astype(jnp.int32)
  dst = edge_index[1].astype(jnp.int32)
  ew = edge_weight.astype(jnp.float32)

  def padded(arrs, lanes, pad_to):
    epw = -(-e // (NW * pad_to)) * pad_to
    pad = epw * NW - e
    pidx = (jnp.arange(pad, dtype=jnp.int32) * 131) % n
    pz = jnp.zeros((pad,), jnp.float32)
    out = []
    for arr in arrs:
      fill = pz if arr.dtype == jnp.float32 else pidx
      out.append(jnp.concatenate([arr, fill])
                 .reshape(NW, epw // lanes, lanes))
    return out, epw // lanes

  (dst_p, ew_p), chd = padded([dst, ew], CW, CW)
  (src_a, dst_a, ew_a), ch = padded([src, dst, ew], ACW, ACW * SCH)

  npad = -(-n // 256) * 256
  a2d = jnp.reshape(a, (1, 1)).astype(jnp.float32)

  degp = _make_deg(npad, chd)(dst_p, ew_p)
  degp2 = degp.reshape(NC, npad // 128, 128)
  dinv2d, sl2d = pl.pallas_call(
      _dinv_tc,
      out_shape=[jax.ShapeDtypeStruct((npad // 128, 128), jnp.float32),
                 jax.ShapeDtypeStruct((npad // 128, 128), jnp.float32)],
  )(degp2)
  dinv = dinv2d.reshape(npad)
  sl_n = sl2d.reshape(npad, 1)

  bn = 1000
  grid = n // bn
  mm = pl.pallas_call(
      _xw_tc,
      grid=(grid,),
      in_specs=[pl.BlockSpec((bn, d), lambda i: (i, 0)),
                pl.BlockSpec((d, d), lambda i: (0, 0))],
      out_specs=pl.BlockSpec((bn, d), lambda i: (i, 0)),
      out_shape=jax.ShapeDtypeStruct((n, d), jnp.float32),
  )
  xw1 = mm(x, W1)

  agg = _make_agg(n, npad, ch, d)
  acc1 = agg(src_a, dst_a, ew_a, dinv, xw1)

  epi_mm = pl.pallas_call(
      _epi_mm_tc,
      grid=(grid,),
      in_specs=[pl.BlockSpec((NC, bn, d), lambda i: (0, i, 0)),
                pl.BlockSpec((bn, d), lambda i: (i, 0)),
                pl.BlockSpec((bn, 1), lambda i: (i, 0)),
                pl.BlockSpec((1, d), lambda i: (0, 0)),
                pl.BlockSpec((d, d), lambda i: (0, 0)),
                pl.BlockSpec((1, 1), lambda i: (0, 0))],
      out_specs=[pl.BlockSpec((bn, d), lambda i: (i, 0)),
                 pl.BlockSpec((bn, d), lambda i: (i, 0))],
      out_shape=[jax.ShapeDtypeStruct((n, d), jnp.float32),
                 jax.ShapeDtypeStruct((n, d), jnp.float32)],
  )
  h1, xw2 = epi_mm(acc1, xw1, sl_n, b1.reshape(1, d), W2, a2d)

  acc2 = agg(src_a, dst_a, ew_a, dinv, xw2)

  epi = pl.pallas_call(
      _epi_tc,
      grid=(grid,),
      in_specs=[pl.BlockSpec((NC, bn, d), lambda i: (0, i, 0)),
                pl.BlockSpec((bn, d), lambda i: (i, 0)),
                pl.BlockSpec((bn, 1), lambda i: (i, 0)),
                pl.BlockSpec((1, d), lambda i: (0, 0)),
                pl.BlockSpec((1, 1), lambda i: (0, 0))],
      out_specs=pl.BlockSpec((bn, d), lambda i: (i, 0)),
      out_shape=jax.ShapeDtypeStruct((n, d), jnp.float32),
  )
  h2 = epi(acc2, xw2, sl_n, b2.reshape(1, d), a2d)
  return h1, h2


# speedup vs baseline: 1.8688x; 1.0256x over previous
import functools

import jax
import jax.numpy as jnp
from jax import lax
from jax.experimental import pallas as pl
from jax.experimental.pallas import tpu as pltpu
from jax.experimental.pallas import tpu_sc as plsc

NC = 2
NS = 16
NW = NC * NS
CW = 128
ACW = 80
SCH = 12

_mesh = functools.partial(
    plsc.VectorSubcoreMesh, core_axis_name="c", subcore_axis_name="s")
_params = pltpu.CompilerParams(needs_layout_passes=False,
                               use_tc_tiling_on_sc=False)
_params_tc = pltpu.CompilerParams(needs_layout_passes=False)


def _deg_body(npad, ch, dst_hbm, ew_hbm, out_hbm, dst_v, ew_v, zv, deg_sp):
  c = lax.axis_index("c")
  s = lax.axis_index("s")
  wid = s * NC + c
  nsl = npad // NS

  def zbody(i, _):
    zv[pl.ds(i * 16, 16)] = jnp.zeros((16,), jnp.float32)
    return ()

  lax.fori_loop(0, nsl // 16, zbody, ())
  pltpu.sync_copy(zv, deg_sp.at[pl.ds(s * nsl, nsl)])
  pltpu.sync_copy(dst_hbm.at[wid], dst_v)
  pltpu.sync_copy(ew_hbm.at[wid], ew_v)
  plsc.subcore_barrier()

  def body(k, _):
    pltpu.sync_copy(ew_v.at[k], deg_sp.at[dst_v.at[k]], add=True)
    return ()

  lax.fori_loop(0, ch, body, ())
  plsc.subcore_barrier()
  pltpu.sync_copy(deg_sp.at[pl.ds(s * nsl, nsl)], zv)
  pltpu.sync_copy(zv, out_hbm.at[pl.ds(c * npad + s * nsl, nsl)])


def _agg_body(n, npad, ch, d, src_hbm, dst_hbm, ew_hbm, dinv_hbm, xw_hbm,
              out_hbm, src_v, dst_v, ew_v, norm_v, dinv_v,
              rows_a, rows_b, rows_c, acc_sp,
              ga, gb, gc, sa, sb_, sc_):
  c = lax.axis_index("c")
  s = lax.axis_index("s")
  wid = s * NC + c
  rsl = npad // NS
  bufs = (rows_a, rows_b, rows_c)
  gsems = (ga, gb, gc)
  ssems = (sa, sb_, sc_)

  def zrow(r, _):
    for j in range(d // 16):
      rows_a[r, pl.ds(j * 16, 16)] = jnp.zeros((16,), jnp.float32)
    return ()

  lax.fori_loop(0, ACW, zrow, ())
  for b in range(rsl // ACW):
    pltpu.sync_copy(rows_a, acc_sp.at[pl.ds(s * rsl + b * ACW, ACW)])
  pltpu.sync_copy(dinv_hbm, dinv_v)
  plsc.subcore_barrier()

  def scale_rows(buf, k):
    k16 = jnp.full((16,), k, jnp.int32)
    unroll = 4

    def row(i, _):
      r = unroll * i
      nbs = [plsc.load_gather(
          norm_v, [k16, jnp.full((16,), r + u, jnp.int32)])
          for u in range(unroll)]
      for u in range(unroll):
        for j in range(d // 16):
          sl = pl.ds(j * 16, 16)
          buf[r + u, sl] = buf[r + u, sl] * nbs[u]
      return ()

    lax.fori_loop(0, ACW // unroll, row, ())

  def gat(k, b):
    return pltpu.make_async_copy(xw_hbm.at[src_v.at[k]], bufs[b], gsems[b])

  def scat(k, b):
    return pltpu.make_async_copy(bufs[b], acc_sp.at[dst_v.at[k]], ssems[b])

  def super_chunk(sb, _):
    pltpu.sync_copy(src_hbm.at[wid, pl.ds(sb * SCH, SCH)], src_v)
    pltpu.sync_copy(dst_hbm.at[wid, pl.ds(sb * SCH, SCH)], dst_v)
    pltpu.sync_copy(ew_hbm.at[wid, pl.ds(sb * SCH, SCH)], ew_v)

    def norm_row(r, _):
      for j in range(ACW // 16):
        sidx = src_v[r, pl.ds(j * 16, 16)]
        didx = dst_v[r, pl.ds(j * 16, 16)]
        dv_s = plsc.load_gather(dinv_v, [sidx])
        dv_d = plsc.load_gather(dinv_v, [didx])
        norm_v[r, pl.ds(j * 16, 16)] = (
            dv_s * ew_v[r, pl.ds(j * 16, 16)] * dv_d)
      return ()

    lax.fori_loop(0, SCH, norm_row, ())

    gat(0, 0).start()
    gat(1, 1).start()
    for k in range(SCH):
      this = k % 3
      other = (k + 2) % 3
      gat(k, this).wait()
      scale_rows(bufs[this], k)
      if k >= 1:
        scat(k - 1, other).wait()
      if k + 2 < SCH:
        gat(k + 2, other).start()
      scat(k, this).start(add=True)
    scat(SCH - 1, (SCH - 1) % 3).wait()
    return ()

  lax.fori_loop(0, ch // SCH, super_chunk, ())
  plsc.subcore_barrier()
  for b in range(rsl // ACW):
    buf = bufs[b % 3]
    pltpu.sync_copy(acc_sp.at[pl.ds(s * rsl + b * ACW, ACW)], buf)
    pltpu.sync_copy(buf, out_hbm.at[c, pl.ds(s * rsl + b * ACW, ACW)])


def _make_deg(npad, ch):
  return pl.kernel(
      functools.partial(_deg_body, npad, ch),
      out_type=jax.ShapeDtypeStruct((NC * npad,), jnp.float32),
      mesh=_mesh(),
      compiler_params=_params,
      scratch_types=[
          pltpu.VMEM((ch, CW), jnp.int32),
          pltpu.VMEM((ch, CW), jnp.float32),
          pltpu.VMEM((npad // NS,), jnp.float32),
          pltpu.VMEM_SHARED((npad,), jnp.float32),
      ])


def _make_agg(n, npad, ch, d):
  return pl.kernel(
      functools.partial(_agg_body, n, npad, ch, d),
      out_type=jax.ShapeDtypeStruct((NC, npad, d), jnp.float32),
      mesh=_mesh(),
      compiler_params=_params,
      scratch_types=[
          pltpu.VMEM((SCH, ACW), jnp.int32),
          pltpu.VMEM((SCH, ACW), jnp.int32),
          pltpu.VMEM((SCH, ACW), jnp.float32),
          pltpu.VMEM((SCH, ACW), jnp.float32),
          pltpu.VMEM((npad,), jnp.float32),
          pltpu.VMEM((ACW, d), jnp.float32),
          pltpu.VMEM((ACW, d), jnp.float32),
          pltpu.VMEM((ACW, d), jnp.float32),
          pltpu.VMEM_SHARED((npad, d), jnp.float32),
          pltpu.SemaphoreType.DMA,
          pltpu.SemaphoreType.DMA,
          pltpu.SemaphoreType.DMA,
          pltpu.SemaphoreType.DMA,
          pltpu.SemaphoreType.DMA,
          pltpu.SemaphoreType.DMA,
      ])


def _dinv_tc(degp_ref, dinv_ref, sl_ref):
  deg = 1.0 + degp_ref[0] + degp_ref[1]
  dinv = jnp.where(deg > 0, lax.rsqrt(jnp.where(deg > 0, deg, 1.0)), 0.0)
  dinv_ref[...] = dinv
  sl_ref[...] = dinv * dinv


def _xw_tc(x_ref, w_ref, out_ref):
  out_ref[...] = lax.dot_general(
      x_ref[...], w_ref[...], (((1,), (0,)), ((), ())),
      preferred_element_type=jnp.float32)


def _epi_mm_tc(acc_ref, xw_ref, sl_ref, b_ref, w_ref, a_ref, h_ref, xw2_ref):
  a = a_ref[0, 0]
  h = acc_ref[0] + acc_ref[1] + xw_ref[...] * sl_ref[...] + b_ref[...]
  h = jnp.where(h >= 0, h, a * h)
  h_ref[...] = h
  xw2_ref[...] = lax.dot_general(
      h, w_ref[...], (((1,), (0,)), ((), ())),
      preferred_element_type=jnp.float32)


def _epi_tc(acc_ref, xw_ref, sl_ref, b_ref, a_ref, h_ref):
  a = a_ref[0, 0]
  h = acc_ref[0] + acc_ref[1] + xw_ref[...] * sl_ref[...] + b_ref[...]
  h_ref[...] = jnp.where(h >= 0, h, a * h)


def kernel(x, edge_index, edge_weight, W1, b1, W2, b2, a):
  n, d = x.shape
  e = edge_weight.shape[0]
  src = edge_index[0].astype(jnp.int32)
  dst = edge_index[1].astype(jnp.int32)
  ew = edge_weight.astype(jnp.float32)

  def padded(arrs, lanes, pad_to):
    epw = -(-e // (NW * pad_to)) * pad_to
    pad = epw * NW - e
    pidx = (jnp.arange(pad, dtype=jnp.int32) * 131) % n
    pz = jnp.zeros((pad,), jnp.float32)
    out = []
    for arr in arrs:
      fill = pz if arr.dtype == jnp.float32 else pidx
      out.append(jnp.concatenate([arr, fill])
                 .reshape(NW, epw // lanes, lanes))
    return out, epw // lanes

  (dst_p, ew_p), chd = padded([dst, ew], CW, CW)
  (src_a, dst_a, ew_a), ch = padded([src, dst, ew], ACW, ACW * SCH)

  npad = -(-n // 256) * 256
  a2d = jnp.reshape(a, (1, 1)).astype(jnp.float32)

  degp = _make_deg(npad, chd)(dst_p, ew_p)
  degp2 = degp.reshape(NC, npad // 128, 128)
  dinv2d, sl2d = pl.pallas_call(
      _dinv_tc,
      out_shape=[jax.ShapeDtypeStruct((npad // 128, 128), jnp.float32),
                 jax.ShapeDtypeStruct((npad // 128, 128), jnp.float32)],
  )(degp2)
  dinv = dinv2d.reshape(npad)
  sl_n = sl2d.reshape(npad, 1)

  bn = 1000
  grid = n // bn
  mm = pl.pallas_call(
      _xw_tc,
      grid=(grid,),
      in_specs=[pl.BlockSpec((bn, d), lambda i: (i, 0)),
                pl.BlockSpec((d, d), lambda i: (0, 0))],
      out_specs=pl.BlockSpec((bn, d), lambda i: (i, 0)),
      out_shape=jax.ShapeDtypeStruct((n, d), jnp.float32),
  )
  xw1 = mm(x, W1)

  agg = _make_agg(n, npad, ch, d)
  acc1 = agg(src_a, dst_a, ew_a, dinv, xw1)

  epi_mm = pl.pallas_call(
      _epi_mm_tc,
      grid=(grid,),
      in_specs=[pl.BlockSpec((NC, bn, d), lambda i: (0, i, 0)),
                pl.BlockSpec((bn, d), lambda i: (i, 0)),
                pl.BlockSpec((bn, 1), lambda i: (i, 0)),
                pl.BlockSpec((1, d), lambda i: (0, 0)),
                pl.BlockSpec((d, d), lambda i: (0, 0)),
                pl.BlockSpec((1, 1), lambda i: (0, 0))],
      out_specs=[pl.BlockSpec((bn, d), lambda i: (i, 0)),
                 pl.BlockSpec((bn, d), lambda i: (i, 0))],
      out_shape=[jax.ShapeDtypeStruct((n, d), jnp.float32),
                 jax.ShapeDtypeStruct((n, d), jnp.float32)],
  )
  h1, xw2 = epi_mm(acc1, xw1, sl_n, b1.reshape(1, d), W2, a2d)

  acc2 = agg(src_a, dst_a, ew_a, dinv, xw2)

  epi = pl.pallas_call(
      _epi_tc,
      grid=(grid,),
      in_specs=[pl.BlockSpec((NC, bn, d), lambda i: (0, i, 0)),
                pl.BlockSpec((bn, d), lambda i: (i, 0)),
                pl.BlockSpec((bn, 1), lambda i: (i, 0)),
                pl.BlockSpec((1, d), lambda i: (0, 0)),
                pl.BlockSpec((1, 1), lambda i: (0, 0))],
      out_specs=pl.BlockSpec((bn, d), lambda i: (i, 0)),
      out_shape=jax.ShapeDtypeStruct((n, d), jnp.float32),
  )
  h2 = epi(acc2, xw2, sl_n, b2.reshape(1, d), a2d)
  return h1, h2
